# Initial kernel scaffold; baseline (speedup 1.0000x reference)
import functools

import jax
import jax.numpy as jnp
from jax import lax
from jax.experimental import pallas as pl
from jax.experimental.pallas import tpu as pltpu
from jax.experimental.pallas import tpu_sc as plsc

N = 10000
E = 320000
C = 128
NC = 2
NS = 16
NW = NC * NS
EPW = E // NW
K = 128
NFULL = EPW // K
TAIL = EPW - NFULL * K
CNTW = 16
STRIPE = 624
STRIPE_LAST = N - (NS - 1) * STRIPE

_MESH = plsc.VectorSubcoreMesh(core_axis_name="c", subcore_axis_name="s")


@functools.partial(
    pl.kernel,
    out_type=(
        jax.ShapeDtypeStruct((NC, N, C), jnp.float32),
        jax.ShapeDtypeStruct((NC, N, CNTW), jnp.float32),
    ),
    mesh=_MESH,
    scratch_types=[
        pltpu.VMEM((K,), jnp.int32),
        pltpu.VMEM((K,), jnp.int32),
        pltpu.VMEM((K, C), jnp.float32),
        pltpu.VMEM((K, CNTW), jnp.float32),
        pltpu.VMEM((TAIL,), jnp.int32),
        pltpu.VMEM((TAIL,), jnp.int32),
        pltpu.VMEM((TAIL, C), jnp.float32),
        pltpu.VMEM_SHARED((N, C), jnp.float32),
        pltpu.VMEM_SHARED((N, CNTW), jnp.float32),
        pltpu.SemaphoreType.DMA,
    ],
)
def _sc_segment_sum(src_hbm, dst_hbm, x_hbm, zrow_hbm, zcnt_hbm, ones_hbm,
                    agg_out, cnt_out, src_v, dst_v, rows_v, ones_v,
                    src_t, dst_t, rows_t, agg_sh, cnt_sh, sem):
    cid = lax.axis_index("c")
    sid = lax.axis_index("s")
    wid = sid * NC + cid

    pltpu.sync_copy(ones_hbm, ones_v)

    @pl.when(sid < NS - 1)
    def _():
        pltpu.sync_copy(zrow_hbm.at[pl.ds(0, STRIPE)],
                        agg_sh.at[pl.ds(sid * STRIPE, STRIPE)])
        pltpu.sync_copy(zcnt_hbm.at[pl.ds(0, STRIPE)],
                        cnt_sh.at[pl.ds(sid * STRIPE, STRIPE)])

    @pl.when(sid == NS - 1)
    def _():
        pltpu.sync_copy(zrow_hbm, agg_sh.at[pl.ds((NS - 1) * STRIPE, STRIPE_LAST)])
        pltpu.sync_copy(zcnt_hbm, cnt_sh.at[pl.ds((NS - 1) * STRIPE, STRIPE_LAST)])

    plsc.subcore_barrier()

    ebase = wid * EPW

    @pl.loop(0, NFULL)
    def _(i):
        base = ebase + i * K
        pltpu.sync_copy(src_hbm.at[pl.ds(base, K)], src_v)
        pltpu.sync_copy(dst_hbm.at[pl.ds(base, K)], dst_v)
        pltpu.async_copy(x_hbm.at[src_v], rows_v, sem).wait()
        pltpu.sync_copy(rows_v, agg_sh.at[dst_v], add=True)
        pltpu.sync_copy(ones_v, cnt_sh.at[dst_v], add=True)

    tbase = ebase + NFULL * K
    pltpu.sync_copy(src_hbm.at[pl.ds(tbase, TAIL)], src_t)
    pltpu.sync_copy(dst_hbm.at[pl.ds(tbase, TAIL)], dst_t)
    pltpu.async_copy(x_hbm.at[src_t], rows_t, sem).wait()
    pltpu.sync_copy(rows_t, agg_sh.at[dst_t], add=True)
    pltpu.sync_copy(ones_v.at[pl.ds(0, TAIL)], cnt_sh.at[dst_t], add=True)

    plsc.subcore_barrier()

    @pl.when(sid < NS - 1)
    def _():
        pltpu.sync_copy(agg_sh.at[pl.ds(sid * STRIPE, STRIPE)],
                        agg_out.at[cid, pl.ds(sid * STRIPE, STRIPE)])
        pltpu.sync_copy(cnt_sh.at[pl.ds(sid * STRIPE, STRIPE)],
                        cnt_out.at[cid, pl.ds(sid * STRIPE, STRIPE)])

    @pl.when(sid == NS - 1)
    def _():
        pltpu.sync_copy(agg_sh.at[pl.ds((NS - 1) * STRIPE, STRIPE_LAST)],
                        agg_out.at[cid, pl.ds((NS - 1) * STRIPE, STRIPE_LAST)])
        pltpu.sync_copy(cnt_sh.at[pl.ds((NS - 1) * STRIPE, STRIPE_LAST)],
                        cnt_out.at[cid, pl.ds((NS - 1) * STRIPE, STRIPE_LAST)])


_BR = 1000


def _dense_body(x_ref, a_ref, c_ref, wlt_ref, bl_ref, wrt_ref, g_ref, b_ref,
                o_ref):
    agg = a_ref[0] + a_ref[1]
    cnt = c_ref[0][:, 0:1] + c_ref[1][:, 0:1]
    agg = agg / jnp.maximum(cnt, 1.0)
    xb = x_ref[...]
    conv = (jnp.dot(agg, wlt_ref[...], preferred_element_type=jnp.float32)
            + jnp.dot(xb, wrt_ref[...], preferred_element_type=jnp.float32)
            + bl_ref[...])
    mean = jnp.mean(conv, axis=-1, keepdims=True)
    cen = conv - mean
    var = jnp.mean(cen * cen, axis=-1, keepdims=True)
    normed = cen * lax.rsqrt(var + 1e-5) * g_ref[...] + b_ref[...]
    o_ref[...] = xb + jnp.maximum(normed, 0.0)


def _dense(x, aggp, cntp, W_lT, b_l, W_rT, ln_gamma, ln_beta):
    grid = (N // _BR,)
    return pl.pallas_call(
        _dense_body,
        grid=grid,
        in_specs=[
            pl.BlockSpec((_BR, C), lambda i: (i, 0)),
            pl.BlockSpec((NC, _BR, C), lambda i: (0, i, 0)),
            pl.BlockSpec((NC, _BR, CNTW), lambda i: (0, i, 0)),
            pl.BlockSpec((C, C), lambda i: (0, 0)),
            pl.BlockSpec((1, C), lambda i: (0, 0)),
            pl.BlockSpec((C, C), lambda i: (0, 0)),
            pl.BlockSpec((1, C), lambda i: (0, 0)),
            pl.BlockSpec((1, C), lambda i: (0, 0)),
        ],
        out_specs=pl.BlockSpec((_BR, C), lambda i: (i, 0)),
        out_shape=jax.ShapeDtypeStruct((N, C), jnp.float32),
    )(x, aggp, cntp, W_lT, b_l.reshape(1, C), W_rT,
      ln_gamma.reshape(1, C), ln_beta.reshape(1, C))


def kernel(x, edge_index, W_l, b_l, W_r, ln_gamma, ln_beta):
    src = edge_index[0].astype(jnp.int32)
    dst = edge_index[1].astype(jnp.int32)
    zrow = jnp.zeros((STRIPE_LAST, C), jnp.float32)
    zcnt = jnp.zeros((STRIPE_LAST, CNTW), jnp.float32)
    ones = jnp.ones((K, CNTW), jnp.float32)
    aggp, cntp = _sc_segment_sum(src, dst, x, zrow, zcnt, ones)
    return _dense(x, aggp, cntp, W_l.T, b_l, W_r.T, ln_gamma, ln_beta)


# speedup vs baseline: 7.5930x; 7.5930x over previous
import dataclasses
import functools

import jax
import jax.numpy as jnp
from jax import lax
from jax.experimental import pallas as pl
from jax.experimental.pallas import tpu as pltpu
from jax.experimental.pallas import tpu_sc as plsc

N = 10000
E = 320000
C = 128
NC = 2
NS = 16
NW = NC * NS
EPW = E // NW
K = 128
NFULL = EPW // K
TAIL = EPW - NFULL * K
NP = 10240
ZCH = 80
NZCH = N // ZCH
L = 16


def _sc_segment_sum(src_hbm, dst_hbm, x_hbm, zrow_hbm, zcnt_hbm,
                    agg_out, cnt_out, src_v, dst_v, rows_v,
                    src_t, dst_t, rows_t, zbuf, cnt_loc, agg_sh, sem):
    cid = lax.axis_index("c")
    sid = lax.axis_index("s")
    wid = sid * NC + cid

    pltpu.sync_copy(zrow_hbm, zbuf)
    pltpu.sync_copy(zcnt_hbm, cnt_loc)

    @pl.loop(sid, NZCH, step=NS)
    def _(k):
        pltpu.sync_copy(zbuf, agg_sh.at[pl.ds(k * ZCH, ZCH)])

    plsc.subcore_barrier()

    ebase = wid * EPW
    ones16 = jnp.ones((L,), jnp.float32)

    @pl.loop(0, NFULL)
    def _(i):
        base = ebase + i * K
        pltpu.sync_copy(src_hbm.at[pl.ds(base, K)], src_v)
        pltpu.sync_copy(dst_hbm.at[pl.ds(base, K)], dst_v)
        pltpu.async_copy(x_hbm.at[src_v], rows_v, sem).wait()
        pltpu.sync_copy(rows_v, agg_sh.at[dst_v], add=True)
        for j in range(K // L):
            plsc.addupdate_scatter(cnt_loc, [dst_v[pl.ds(j * L, L)]], ones16)

    tbase = ebase + NFULL * K
    pltpu.sync_copy(src_hbm.at[pl.ds(tbase, TAIL)], src_t)
    pltpu.sync_copy(dst_hbm.at[pl.ds(tbase, TAIL)], dst_t)
    pltpu.async_copy(x_hbm.at[src_t], rows_t, sem).wait()
    pltpu.sync_copy(rows_t, agg_sh.at[dst_t], add=True)
    for j in range(TAIL // L):
        plsc.addupdate_scatter(cnt_loc, [dst_t[pl.ds(j * L, L)]], ones16)

    pltpu.sync_copy(cnt_loc, cnt_out.at[pl.ds(wid * NP, NP)])

    plsc.subcore_barrier()

    @pl.loop(sid, NZCH, step=NS)
    def _(k):
        r0 = k * ZCH
        pltpu.sync_copy(agg_sh.at[pl.ds(r0, ZCH)], zbuf)
        pltpu.sync_copy(zbuf, agg_out.at[pl.ds(cid * N + r0, ZCH)])


@functools.cache
def _sc_segment_sum_call():
    mesh = plsc.VectorSubcoreMesh(core_axis_name="c", subcore_axis_name="s",
                                  num_cores=NC, num_subcores=NS)
    cp = pltpu.CompilerParams()
    if "needs_layout_passes" in pltpu.CompilerParams.__dataclass_fields__:
        cp = dataclasses.replace(cp, needs_layout_passes=False)
    return pl.kernel(
        _sc_segment_sum,
        out_type=(
            jax.ShapeDtypeStruct((NC * N, C), jnp.float32),
            jax.ShapeDtypeStruct((NW * NP,), jnp.float32),
        ),
        mesh=mesh,
        compiler_params=cp,
        scratch_types=[
            pltpu.VMEM((K,), jnp.int32),
            pltpu.VMEM((K,), jnp.int32),
            pltpu.VMEM((K, C), jnp.float32),
            pltpu.VMEM((TAIL,), jnp.int32),
            pltpu.VMEM((TAIL,), jnp.int32),
            pltpu.VMEM((TAIL, C), jnp.float32),
            pltpu.VMEM((ZCH, C), jnp.float32),
            pltpu.VMEM((NP,), jnp.float32),
            pltpu.VMEM_SHARED((N, C), jnp.float32),
            pltpu.SemaphoreType.DMA,
        ],
    )


_BR = 1000
_NBLK = N // _BR


def _dense_body(x_ref, a0_ref, a1_ref, c_ref, wlt_ref, bl_ref,
                wrt_ref, g_ref, b_ref, o_ref):
    cnt_row = jnp.sum(c_ref[0], axis=0, keepdims=True)
    recip_row = 1.0 / jnp.maximum(cnt_row, 1.0)
    rows_i = lax.broadcasted_iota(jnp.int32, (_BR, _BR), 0)
    cols_i = lax.broadcasted_iota(jnp.int32, (_BR, _BR), 1)
    diag = jnp.where(rows_i == cols_i,
                     jnp.broadcast_to(recip_row, (_BR, _BR)), 0.0)
    agg = a0_ref[...] + a1_ref[...]
    aggm = jnp.dot(diag, agg, preferred_element_type=jnp.float32)
    xb = x_ref[...]
    conv = (jnp.dot(aggm, wlt_ref[...], preferred_element_type=jnp.float32)
            + jnp.dot(xb, wrt_ref[...], preferred_element_type=jnp.float32)
            + bl_ref[...])
    mean = jnp.mean(conv, axis=-1, keepdims=True)
    cen = conv - mean
    var = jnp.mean(cen * cen, axis=-1, keepdims=True)
    normed = cen * lax.rsqrt(var + 1e-5) * g_ref[...] + b_ref[...]
    o_ref[...] = xb + jnp.maximum(normed, 0.0)


def _dense(x, aggp, cntp, W_lT, b_l, W_rT, ln_gamma, ln_beta):
    return pl.pallas_call(
        _dense_body,
        grid=(_NBLK,),
        in_specs=[
            pl.BlockSpec((_BR, C), lambda i: (i, 0)),
            pl.BlockSpec((_BR, C), lambda i: (i, 0)),
            pl.BlockSpec((_BR, C), lambda i: (i + _NBLK, 0)),
            pl.BlockSpec((1, NW, _BR), lambda i: (i, 0, 0)),
            pl.BlockSpec((C, C), lambda i: (0, 0)),
            pl.BlockSpec((1, C), lambda i: (0, 0)),
            pl.BlockSpec((C, C), lambda i: (0, 0)),
            pl.BlockSpec((1, C), lambda i: (0, 0)),
            pl.BlockSpec((1, C), lambda i: (0, 0)),
        ],
        out_specs=pl.BlockSpec((_BR, C), lambda i: (i, 0)),
        out_shape=jax.ShapeDtypeStruct((N, C), jnp.float32),
    )(x, aggp, aggp, cntp, W_lT, b_l.reshape(1, C), W_rT,
      ln_gamma.reshape(1, C), ln_beta.reshape(1, C))


def kernel(x, edge_index, W_l, b_l, W_r, ln_gamma, ln_beta):
    src = edge_index[0].astype(jnp.int32)
    dst = edge_index[1].astype(jnp.int32)
    zrow = jnp.zeros((ZCH, C), jnp.float32)
    zcnt = jnp.zeros((NP,), jnp.float32)
    aggp, cntp = _sc_segment_sum_call()(src, dst, x, zrow, zcnt)
    cnt3d = cntp.reshape(NW, NP)[:, :N].reshape(NW, _NBLK, _BR).transpose(1, 0, 2)
    return _dense(x, aggp, cnt3d, W_l.T, b_l, W_r.T, ln_gamma, ln_beta)
